# baseline (speedup 1.0000x reference)
import functools

import jax
import jax.numpy as jnp
from jax.experimental import pallas as pl
from jax.experimental.pallas import tpu as pltpu

_K = 16
_QT = 1024
_NT = 2048


def _knn_body(q_ref, pt_ref, dist_ref, idx_ref, d2_ref, tv_ref, ti_ref,
              *, nt, n_chunks):
    j = pl.program_id(1)
    qt = q_ref.shape[0]
    nb = nt // 128

    @pl.when(j == 0)
    def _init():
        tv_ref[...] = jnp.full(tv_ref.shape, jnp.inf, dtype=tv_ref.dtype)
        ti_ref[...] = jnp.zeros(ti_ref.shape, dtype=ti_ref.dtype)

    q = q_ref[...]
    pt = pt_ref[...]
    qsq = jnp.sum(q * q, axis=1, keepdims=True)
    psq = jnp.sum(pt * pt, axis=0, keepdims=True)
    qp = jnp.dot(q, pt, preferred_element_type=jnp.float32)
    d2 = (qsq + psq) - 2.0 * qp

    base = j * nt
    i128 = jax.lax.broadcasted_iota(jnp.int32, (qt, 128), 1)
    biota = jax.lax.broadcasted_iota(jnp.int32, (qt, nb), 1)
    col = jax.lax.broadcasted_iota(jnp.int32, tv_ref.shape, 1)

    bms = []
    for b in range(nb):
        s = d2[:, b * 128:(b + 1) * 128]
        d2_ref[:, b * 128:(b + 1) * 128] = s
        bms.append(jnp.min(s, axis=1, keepdims=True))
    bm0 = jnp.concatenate(bms, axis=1)

    go = jnp.any(bm0 < tv_ref[:, _K - 1:_K])

    @pl.when(go)
    def _rounds():
        def r_cond(carry):
            return carry[0]

        def r_body(carry):
            _, t1 = carry
            bms, bls = [], []
            for b in range(nb):
                s = d2_ref[:, b * 128:(b + 1) * 128]
                s = jnp.where(i128 == t1[:, b:b + 1], jnp.inf, s)
                d2_ref[:, b * 128:(b + 1) * 128] = s
                mb = jnp.min(s, axis=1, keepdims=True)
                lb = jnp.min(jnp.where(s == mb, i128, 128), axis=1,
                             keepdims=True)
                bms.append(mb)
                bls.append(lb)
            bm = jnp.concatenate(bms, axis=1)
            bl = jnp.concatenate(bls, axis=1)

            def s_cond(sc):
                return sc[0]

            def s_body(sc):
                _, bm, t1, ext = sc
                thr = tv_ref[:, _K - 1:_K]
                v = jnp.min(bm, axis=1, keepdims=True)
                act = v < thr
                bsel = jnp.min(jnp.where(bm == v, biota, nb), axis=1,
                               keepdims=True)
                onb = biota == bsel
                lsel = jnp.min(jnp.where(onb, bl, 128), axis=1,
                               keepdims=True)
                gidx = base + bsel * 128 + lsel
                tv = tv_ref[...]
                ti = ti_ref[...]
                pos = jnp.sum(((tv < v) | ((tv == v) & (ti < gidx)))
                              .astype(jnp.int32), axis=1, keepdims=True)
                tv_s = jnp.concatenate([tv[:, :1], tv[:, :-1]], axis=1)
                ti_s = jnp.concatenate([ti[:, :1], ti[:, :-1]], axis=1)
                ntv = jnp.where(col < pos, tv,
                                jnp.where(col == pos, v, tv_s))
                nti = jnp.where(col < pos, ti,
                                jnp.where(col == pos, gidx, ti_s))
                tv_ref[...] = jnp.where(act, ntv, tv)
                ti_ref[...] = jnp.where(act, nti, ti)
                upd = onb & act
                bm = jnp.where(upd, jnp.inf, bm)
                t1 = jnp.where(upd, bl, t1)
                ext = ext | jnp.any(act)
                cont = jnp.any(jnp.min(bm, axis=1, keepdims=True)
                               < tv_ref[:, _K - 1:_K])
                return cont, bm, t1, ext

            cont0 = jnp.any(bm < tv_ref[:, _K - 1:_K])
            _, _, t1n, extn = jax.lax.while_loop(
                s_cond, s_body,
                (cont0, bm, jnp.full((qt, nb), -1, jnp.int32),
                 jnp.full((), False)))
            return extn, t1n

        jax.lax.while_loop(r_cond, r_body,
                           (go, jnp.full((qt, nb), -1, jnp.int32)))

    @pl.when(j == n_chunks - 1)
    def _write():
        dist_ref[...] = tv_ref[...]
        idx_ref[...] = ti_ref[...]


def kernel(pointcloud, query_points, k):
    B, Q, _ = query_points.shape
    N = pointcloud.shape[0] * pointcloud.shape[1]
    p = pointcloud.reshape(-1, 3)
    q = query_points.reshape(-1, 3)

    nt = _NT
    n_pad = ((N + nt - 1) // nt) * nt
    pt = jnp.concatenate(
        [p.T, jnp.full((3, n_pad - N), 1e5, dtype=p.dtype)], axis=1)

    qt = min(_QT, Q)
    n_chunks = n_pad // nt
    grid = (Q // qt, n_chunks)

    dist, idx = pl.pallas_call(
        functools.partial(_knn_body, nt=nt, n_chunks=n_chunks),
        grid=grid,
        in_specs=[
            pl.BlockSpec((qt, 3), lambda i, j: (i, 0)),
            pl.BlockSpec((3, nt), lambda i, j: (0, j)),
        ],
        out_specs=[
            pl.BlockSpec((qt, _K), lambda i, j: (i, 0)),
            pl.BlockSpec((qt, _K), lambda i, j: (i, 0)),
        ],
        out_shape=[
            jax.ShapeDtypeStruct((Q, _K), jnp.float32),
            jax.ShapeDtypeStruct((Q, _K), jnp.int32),
        ],
        scratch_shapes=[
            pltpu.VMEM((qt, nt), jnp.float32),
            pltpu.VMEM((qt, _K), jnp.float32),
            pltpu.VMEM((qt, _K), jnp.int32),
        ],
    )(q, pt)

    return dist.reshape(B, Q, _K), idx.reshape(B, Q, _K)


# speedup vs baseline: 2.4383x; 2.4383x over previous
import functools

import jax
import jax.numpy as jnp
from jax.experimental import pallas as pl
from jax.experimental.pallas import tpu as pltpu

_K = 16
_QT = 1024
_NT = 4096


def _knn_body(q_ref, pt_ref, dist_ref, idx_ref, d2_ref, tv_ref, ti_ref,
              *, nt, n_chunks):
    j = pl.program_id(1)
    qt = q_ref.shape[0]
    n_tiles = nt // 128

    @pl.when(j == 0)
    def _init():
        tv_ref[...] = jnp.full(tv_ref.shape, jnp.inf, dtype=tv_ref.dtype)
        ti_ref[...] = jnp.zeros(ti_ref.shape, dtype=ti_ref.dtype)

    q = q_ref[...]
    pt = pt_ref[...]
    qsq = jnp.sum(q * q, axis=1, keepdims=True)
    psq = jnp.sum(pt * pt, axis=0, keepdims=True)
    qp = jnp.dot(q, pt, preferred_element_type=jnp.float32)
    d2 = (qsq + psq) - 2.0 * qp

    base = j * nt
    i128 = jax.lax.broadcasted_iota(jnp.int32, (qt, 128), 1)
    col = jax.lax.broadcasted_iota(jnp.int32, tv_ref.shape, 1)

    red0 = d2[:, 0:128]
    d2_ref[:, 0:128] = red0
    for t in range(1, n_tiles):
        s = d2[:, t * 128:(t + 1) * 128]
        d2_ref[:, t * 128:(t + 1) * 128] = s
        red0 = jnp.minimum(red0, s)

    go = jnp.any(red0 < tv_ref[:, _K - 1:_K])

    @pl.when(go)
    def _rounds():
        def r_cond(carry):
            return carry[0]

        def r_body(carry):
            _, taken0 = carry
            red = jnp.where(taken0 == 0, jnp.inf, d2_ref[:, 0:128])
            d2_ref[:, 0:128] = red
            att = jnp.zeros((qt, 128), jnp.int32)
            for t in range(1, n_tiles):
                s = d2_ref[:, t * 128:(t + 1) * 128]
                s = jnp.where(taken0 == t, jnp.inf, s)
                d2_ref[:, t * 128:(t + 1) * 128] = s
                upd = s < red
                red = jnp.where(upd, s, red)
                att = jnp.where(upd, t, att)

            def s_cond(sc):
                return sc[0]

            def s_body(sc):
                _, red, taken, ext = sc
                thr = tv_ref[:, _K - 1:_K]
                v = jnp.min(red, axis=1, keepdims=True)
                act = v < thr
                lsel = jnp.min(jnp.where(red == v, i128, 128), axis=1,
                               keepdims=True)
                on = i128 == lsel
                tsel = jnp.min(jnp.where(on, att, n_tiles), axis=1,
                               keepdims=True)
                gidx = base + tsel * 128 + lsel
                tv = tv_ref[...]
                ti = ti_ref[...]
                pos = jnp.sum(((tv < v) | ((tv == v) & (ti < gidx)))
                              .astype(jnp.int32), axis=1, keepdims=True)
                tv_s = jnp.concatenate([tv[:, :1], tv[:, :-1]], axis=1)
                ti_s = jnp.concatenate([ti[:, :1], ti[:, :-1]], axis=1)
                ntv = jnp.where(col < pos, tv,
                                jnp.where(col == pos, v, tv_s))
                nti = jnp.where(col < pos, ti,
                                jnp.where(col == pos, gidx, ti_s))
                tv_ref[...] = jnp.where(act, ntv, tv)
                ti_ref[...] = jnp.where(act, nti, ti)
                upd = on & act
                red = jnp.where(upd, jnp.inf, red)
                taken = jnp.where(upd, tsel, taken)
                ext = ext | jnp.any(act)
                cont = jnp.any(red < tv_ref[:, _K - 1:_K])
                return cont, red, taken, ext

            cont0 = jnp.any(red < tv_ref[:, _K - 1:_K])
            _, _, taken_n, ext_n = jax.lax.while_loop(
                s_cond, s_body,
                (cont0, red, jnp.full((qt, 128), -1, jnp.int32),
                 jnp.full((), False)))
            return ext_n, taken_n

        jax.lax.while_loop(r_cond, r_body,
                           (go, jnp.full((qt, 128), -1, jnp.int32)))

    @pl.when(j == n_chunks - 1)
    def _write():
        dist_ref[...] = tv_ref[...]
        idx_ref[...] = ti_ref[...]


def kernel(pointcloud, query_points, k):
    B, Q, _ = query_points.shape
    N = pointcloud.shape[0] * pointcloud.shape[1]
    p = pointcloud.reshape(-1, 3)
    q = query_points.reshape(-1, 3)

    nt = _NT
    n_pad = ((N + nt - 1) // nt) * nt
    pt = jnp.concatenate(
        [p.T, jnp.full((3, n_pad - N), 1e5, dtype=p.dtype)], axis=1)

    qt = min(_QT, Q)
    n_chunks = n_pad // nt
    grid = (Q // qt, n_chunks)

    dist, idx = pl.pallas_call(
        functools.partial(_knn_body, nt=nt, n_chunks=n_chunks),
        grid=grid,
        in_specs=[
            pl.BlockSpec((qt, 3), lambda i, j: (i, 0)),
            pl.BlockSpec((3, nt), lambda i, j: (0, j)),
        ],
        out_specs=[
            pl.BlockSpec((qt, _K), lambda i, j: (i, 0)),
            pl.BlockSpec((qt, _K), lambda i, j: (i, 0)),
        ],
        out_shape=[
            jax.ShapeDtypeStruct((Q, _K), jnp.float32),
            jax.ShapeDtypeStruct((Q, _K), jnp.int32),
        ],
        scratch_shapes=[
            pltpu.VMEM((qt, nt), jnp.float32),
            pltpu.VMEM((qt, _K), jnp.float32),
            pltpu.VMEM((qt, _K), jnp.int32),
        ],
    )(q, pt)

    return dist.reshape(B, Q, _K), idx.reshape(B, Q, _K)


# speedup vs baseline: 4.7555x; 1.9503x over previous
import functools

import jax
import jax.numpy as jnp
from jax.experimental import pallas as pl
from jax.experimental.pallas import tpu as pltpu

_K = 16
_QT = 1024
_NT = 4096
_G = 32


def _knn_body(p_ref, qt_ref, dist_ref, idx_ref, d2_ref, tv_ref, ti_ref,
              *, nt, n_chunks):
    j = pl.program_id(1)
    qt = qt_ref.shape[1]
    n_slabs = nt // _G

    @pl.when(j == 0)
    def _init():
        tv_ref[...] = jnp.full(tv_ref.shape, jnp.inf, dtype=tv_ref.dtype)
        ti_ref[...] = jnp.zeros(ti_ref.shape, dtype=ti_ref.dtype)

    p = p_ref[...]
    qc = qt_ref[...]
    psq = jnp.sum(p * p, axis=1, keepdims=True)
    qsq = jnp.sum(qc * qc, axis=0, keepdims=True)
    qp = jnp.dot(p, qc, preferred_element_type=jnp.float32)
    d2 = (qsq + psq) - 2.0 * qp

    base = j * nt
    giota = jax.lax.broadcasted_iota(jnp.int32, (_G, qt), 0)
    rowi = jax.lax.broadcasted_iota(jnp.int32, (_K, qt), 0)

    red0 = d2[0:_G, :]
    d2_ref[0:_G, :] = red0
    for s in range(1, n_slabs):
        sl = d2[s * _G:(s + 1) * _G, :]
        d2_ref[s * _G:(s + 1) * _G, :] = sl
        red0 = jnp.minimum(red0, sl)

    go = jnp.any(red0 <= tv_ref[_K - 1:_K, :])

    @pl.when(go)
    def _rounds():
        def r_cond(carry):
            return carry[0]

        def r_body(carry):
            _, taken0 = carry
            red = jnp.where(taken0 == 0, jnp.inf, d2_ref[0:_G, :])
            d2_ref[0:_G, :] = red
            att = jnp.zeros((_G, qt), jnp.int32)
            for s in range(1, n_slabs):
                sl = d2_ref[s * _G:(s + 1) * _G, :]
                sl = jnp.where(taken0 == s, jnp.inf, sl)
                d2_ref[s * _G:(s + 1) * _G, :] = sl
                upd = sl < red
                red = jnp.where(upd, sl, red)
                att = jnp.where(upd, s, att)
            gidmap = base + att * _G + giota

            def s_cond(sc):
                return sc[0]

            def s_body(sc):
                _, red, taken, ext = sc
                thr = tv_ref[_K - 1:_K, :]
                tik = ti_ref[_K - 1:_K, :]
                v = jnp.min(red, axis=0, keepdims=True)
                ssel = jnp.min(jnp.where(red == v, giota, _G), axis=0,
                               keepdims=True)
                on = giota == ssel
                gidx = jnp.min(jnp.where(on, gidmap, jnp.int32(2**30)),
                               axis=0, keepdims=True)
                act = (v < thr) | ((v == thr) & (gidx < tik))
                tv = tv_ref[...]
                ti = ti_ref[...]
                pos = jnp.sum(((tv < v) | ((tv == v) & (ti < gidx)))
                              .astype(jnp.int32), axis=0, keepdims=True)
                tv_s = jnp.concatenate([tv[0:1], tv[:-1]], axis=0)
                ti_s = jnp.concatenate([ti[0:1], ti[:-1]], axis=0)
                ntv = jnp.where(rowi < pos, tv,
                                jnp.where(rowi == pos, v, tv_s))
                nti = jnp.where(rowi < pos, ti,
                                jnp.where(rowi == pos, gidx, ti_s))
                tv_ref[...] = jnp.where(act, ntv, tv)
                ti_ref[...] = jnp.where(act, nti, ti)
                upd = on & act
                red = jnp.where(upd, jnp.inf, red)
                taken = jnp.where(upd, att, taken)
                ext = ext | jnp.any(act)
                thr2 = tv_ref[_K - 1:_K, :]
                tik2 = ti_ref[_K - 1:_K, :]
                cont = jnp.any((red < thr2)
                               | ((red == thr2) & (gidmap < tik2)))
                return cont, red, taken, ext

            thr = tv_ref[_K - 1:_K, :]
            tik = ti_ref[_K - 1:_K, :]
            cont0 = jnp.any((red < thr) | ((red == thr) & (gidmap < tik)))
            _, _, taken_n, ext_n = jax.lax.while_loop(
                s_cond, s_body,
                (cont0, red, jnp.full((_G, qt), -1, jnp.int32),
                 jnp.full((), False)))
            return ext_n, taken_n

        jax.lax.while_loop(r_cond, r_body,
                           (go, jnp.full((_G, qt), -1, jnp.int32)))

    @pl.when(j == n_chunks - 1)
    def _write():
        dist_ref[...] = tv_ref[...]
        idx_ref[...] = ti_ref[...]


def kernel(pointcloud, query_points, k):
    B, Q, _ = query_points.shape
    N = pointcloud.shape[0] * pointcloud.shape[1]
    p = pointcloud.reshape(-1, 3)
    q = query_points.reshape(-1, 3)

    nt = _NT
    n_pad = ((N + nt - 1) // nt) * nt
    pp = jnp.concatenate(
        [p, jnp.full((n_pad - N, 3), 1e5, dtype=p.dtype)], axis=0)
    qT = q.T

    qtile = min(_QT, Q)
    n_chunks = n_pad // nt
    grid = (Q // qtile, n_chunks)

    distT, idxT = pl.pallas_call(
        functools.partial(_knn_body, nt=nt, n_chunks=n_chunks),
        grid=grid,
        in_specs=[
            pl.BlockSpec((nt, 3), lambda i, j: (j, 0)),
            pl.BlockSpec((3, qtile), lambda i, j: (0, i)),
        ],
        out_specs=[
            pl.BlockSpec((_K, qtile), lambda i, j: (0, i)),
            pl.BlockSpec((_K, qtile), lambda i, j: (0, i)),
        ],
        out_shape=[
            jax.ShapeDtypeStruct((_K, Q), jnp.float32),
            jax.ShapeDtypeStruct((_K, Q), jnp.int32),
        ],
        scratch_shapes=[
            pltpu.VMEM((nt, qtile), jnp.float32),
            pltpu.VMEM((_K, qtile), jnp.float32),
            pltpu.VMEM((_K, qtile), jnp.int32),
        ],
    )(pp, qT)

    return (distT.T.reshape(B, Q, _K), idxT.T.reshape(B, Q, _K))
